# Initial kernel scaffold; baseline (speedup 1.0000x reference)
import jax
import jax.numpy as jnp
from jax.experimental import pallas as pl


def kernel(x, edge_index, edge_type, W_in, b_in, rel_weight, root, bias, W1, b1, W2, b2):
    raise NotImplementedError("write your pallas kernel here")


# speedup vs baseline: 5.2169x; 5.2169x over previous
import functools

import jax
import jax.numpy as jnp
from jax import lax
from jax.experimental import pallas as pl
from jax.experimental.pallas import tpu as pltpu
from jax.experimental.pallas import tpu_sc as plsc

N = 10000
NP = 10240
DIN = 239
DINP = 256
D = 128
R = 4
C = 2
DC = 32
NCHUNK = D // DC
E = 320000
BLK = 128
NSUB = 16
NCORE = 2
NBLK = 157
EPT = NBLK * BLK
EP = EPT * NSUB
NSEG = R * NP
RPT = NSEG // NSUB
CPY = 512

_f32 = jnp.float32
_i32 = jnp.int32


def _stage1_body(x_ref, w_ref, b_ref, h_ref, h4_ref):
    h = jnp.dot(x_ref[...], w_ref[...], preferred_element_type=_f32)
    h = h + b_ref[...]
    h = jnp.where(h > 0, h, 0.01 * h)
    h_ref[...] = h
    for c in range(NCHUNK):
        h4_ref[c] = h[:, c * DC:(c + 1) * DC]


def _stage1(xp, wp, b):
    bn = 512
    grid = NP // bn
    return pl.pallas_call(
        _stage1_body,
        grid=(grid,),
        in_specs=[
            pl.BlockSpec((bn, DINP), lambda i: (i, 0)),
            pl.BlockSpec((DINP, D), lambda i: (0, 0)),
            pl.BlockSpec((1, D), lambda i: (0, 0)),
        ],
        out_specs=[
            pl.BlockSpec((bn, D), lambda i: (i, 0)),
            pl.BlockSpec((NCHUNK, bn, DC), lambda i: (0, i, 0)),
        ],
        out_shape=[
            jax.ShapeDtypeStruct((NP, D), _f32),
            jax.ShapeDtypeStruct((NCHUNK, NP, DC), _f32),
        ],
    )(xp, wp, b)


def _sc_body(h4, srcp, dstp, typp, sums_out, cnt_out,
             acc, cntacc, src_v, dst_v, typ_v, gidx_v, seg_v, segc_v,
             rows_v, ones_v, zbuf, zflat, cbuf, cflat, sem):
    core = lax.axis_index("c")
    sub = lax.axis_index("s")
    zv = jnp.zeros((16,), _f32)

    for i in range(BLK // 16):
        ones_v[pl.ds(i * 16, 16)] = zv + 1.0

    def _zb(i, carry):
        zbuf[i, pl.ds(0, 16)] = zv
        zbuf[i, pl.ds(16, 16)] = zv
        return carry
    lax.fori_loop(0, CPY, _zb, 0)

    def _zf(i, carry):
        zflat[pl.ds(i * 16, 16)] = zv
        return carry
    lax.fori_loop(0, RPT // 16, _zf, 0)

    for rd in range(2):
        chunk = rd * NCORE + core
        gbase = chunk * NP

        for k in range(RPT // CPY):
            pltpu.sync_copy(zbuf, acc.at[pl.ds(sub * RPT + k * CPY, CPY)])
        if rd == 0:
            @pl.when(core == 0)
            def _():
                pltpu.sync_copy(zflat, cntacc.at[pl.ds(sub * RPT, RPT)])
        plsc.subcore_barrier()

        def _edge_block(b, carry):
            e0 = sub * EPT + b * BLK
            pltpu.sync_copy(srcp.at[pl.ds(e0, BLK)], src_v)
            pltpu.sync_copy(dstp.at[pl.ds(e0, BLK)], dst_v)
            pltpu.sync_copy(typp.at[pl.ds(e0, BLK)], typ_v)
            for i in range(BLK // 16):
                sl = pl.ds(i * 16, 16)
                s_ = src_v[sl]
                d_ = dst_v[sl]
                t_ = typ_v[sl]
                gidx_v[sl] = s_ + gbase
                seg_v[sl] = t_ * NP + d_
                if rd == 0:
                    segc_v[sl] = d_ * R + t_
            pltpu.async_copy(h4.at[gidx_v], rows_v, sem).wait()
            pltpu.sync_copy(rows_v, acc.at[seg_v], add=True)
            if rd == 0:
                @pl.when(core == 0)
                def _():
                    pltpu.sync_copy(ones_v, cntacc.at[segc_v], add=True)
            return carry
        lax.fori_loop(0, NBLK, _edge_block, 0)

        plsc.subcore_barrier()

        for k in range(RPT // CPY):
            r0 = sub * RPT + k * CPY
            pltpu.sync_copy(acc.at[pl.ds(r0, CPY)], cbuf)
            pltpu.sync_copy(cbuf, sums_out.at[chunk, pl.ds(r0, CPY)])
        if rd == 0:
            @pl.when(core == 0)
            def _():
                pltpu.sync_copy(cntacc.at[pl.ds(sub * RPT, RPT)], cflat)
                pltpu.sync_copy(cflat, cnt_out.at[pl.ds(sub * RPT, RPT)])
        plsc.subcore_barrier()


def _sc_agg(h4f, srcp, dstp, typp):
    mesh = plsc.VectorSubcoreMesh(
        core_axis_name="c", subcore_axis_name="s",
        num_cores=NCORE, num_subcores=NSUB)
    return pl.kernel(
        _sc_body,
        out_type=[
            jax.ShapeDtypeStruct((NCHUNK, NSEG, DC), _f32),
            jax.ShapeDtypeStruct((NSEG,), _f32),
        ],
        mesh=mesh,
        compiler_params=pltpu.CompilerParams(use_tc_tiling_on_sc=False),
        scratch_types=[
            pltpu.VMEM_SHARED((NSEG, DC), _f32),
            pltpu.VMEM_SHARED((NSEG,), _f32),
            pltpu.VMEM((BLK,), _i32),
            pltpu.VMEM((BLK,), _i32),
            pltpu.VMEM((BLK,), _i32),
            pltpu.VMEM((BLK,), _i32),
            pltpu.VMEM((BLK,), _i32),
            pltpu.VMEM((BLK,), _i32),
            pltpu.VMEM((BLK, DC), _f32),
            pltpu.VMEM((BLK,), _f32),
            pltpu.VMEM((CPY, DC), _f32),
            pltpu.VMEM((RPT,), _f32),
            pltpu.VMEM((CPY, DC), _f32),
            pltpu.VMEM((RPT,), _f32),
            pltpu.SemaphoreType.DMA,
        ],
    )(h4f, srcp, dstp, typp)


def _stage3_body(h_ref, sums_ref, cnt_ref, root_ref, bias_ref, wf_ref,
                 w1_ref, b1_ref, w2_ref, b2_ref, out_ref):
    t = jnp.dot(h_ref[...], root_ref[...], preferred_element_type=_f32)
    t = t + bias_ref[...]
    rec = 1.0 / jnp.maximum(cnt_ref[...], 1.0)
    pieces = []
    for r in range(R):
        rcol = rec[:, r:r + 1]
        for c in range(NCHUNK):
            pieces.append(sums_ref[c, r] * rcol)
    msg = jnp.concatenate(pieces, axis=1)
    t = t + jnp.dot(msg, wf_ref[...], preferred_element_type=_f32)
    u = jnp.dot(t, w1_ref[...], preferred_element_type=_f32) + b1_ref[...]
    u = jnp.where(u > 0, u, 0.01 * u)
    y = jnp.dot(u, w2_ref[...], preferred_element_type=_f32) + b2_ref[...]
    out_ref[...] = y


def _stage3(h, sums4, cnt2, root, bias, wflat, w1, b1, w2, b2):
    bn = 512
    grid = NP // bn
    return pl.pallas_call(
        _stage3_body,
        grid=(grid,),
        in_specs=[
            pl.BlockSpec((bn, D), lambda i: (i, 0)),
            pl.BlockSpec((NCHUNK, R, bn, DC), lambda i: (0, 0, i, 0)),
            pl.BlockSpec((bn, R), lambda i: (i, 0)),
            pl.BlockSpec((D, D), lambda i: (0, 0)),
            pl.BlockSpec((1, D), lambda i: (0, 0)),
            pl.BlockSpec((R * D, D), lambda i: (0, 0)),
            pl.BlockSpec((D, D), lambda i: (0, 0)),
            pl.BlockSpec((1, D), lambda i: (0, 0)),
            pl.BlockSpec((D, C), lambda i: (0, 0)),
            pl.BlockSpec((1, C), lambda i: (0, 0)),
        ],
        out_specs=pl.BlockSpec((bn, C), lambda i: (i, 0)),
        out_shape=jax.ShapeDtypeStruct((NP, C), _f32),
    )(h, sums4, cnt2, root, bias, wflat, w1, b1, w2, b2)


def kernel(x, edge_index, edge_type, W_in, b_in, rel_weight, root, bias,
           W1, b1, W2, b2):
    xp = jnp.pad(x, ((0, NP - N), (0, DINP - DIN)))
    wp = jnp.pad(W_in, ((0, DINP - DIN), (0, 0)))
    h, h4 = _stage1(xp, wp, b_in.reshape(1, D))
    h4f = h4.reshape(NCHUNK * NP, DC)

    npad = EP - E
    ar = jnp.arange(npad, dtype=_i32)
    srcp = jnp.concatenate([edge_index[0], ar % N])
    dstp = jnp.concatenate([edge_index[1], N + ar % (NP - N)])
    typp = jnp.concatenate([edge_type, ar % R])

    sums, cnt = _sc_agg(h4f, srcp, dstp, typp)
    sums4 = sums.reshape(NCHUNK, R, NP, DC)
    cnt2 = cnt.reshape(NP, R)

    out = _stage3(h, sums4, cnt2, root, bias.reshape(1, D),
                  rel_weight.reshape(R * D, D), W1, b1.reshape(1, D),
                  W2, b2.reshape(1, C))
    return out[:N]
